# baseline (speedup 1.0000x reference)
import functools

import jax
import jax.numpy as jnp
from jax import lax
from jax.experimental import pallas as pl
from jax.experimental.pallas import tpu as pltpu
from jax.experimental.pallas import tpu_sc as plsc

D_MODEL = 64
SCALE = 8.0
NUM_CORES = 2
NUM_SUBCORES = 16
NUM_WORKERS = NUM_CORES * NUM_SUBCORES
VOCAB = 1000000
NPAIR = VOCAB // 2
NBLK = VOCAB // 128
BLK_PER_W = NBLK // NUM_WORKERS + 1

_MESH = plsc.VectorSubcoreMesh(core_axis_name="c", subcore_axis_name="s")
_PARAMS = pltpu.CompilerParams(
    use_tc_tiling_on_sc=True, needs_layout_passes=False
)


def _worker_id():
    return lax.axis_index("s") * NUM_CORES + lax.axis_index("c")


@functools.partial(
    pl.kernel,
    mesh=_MESH,
    out_type=jax.ShapeDtypeStruct((NPAIR, 128), jnp.float32),
    compiler_params=_PARAMS,
    scratch_types=[
        pltpu.VMEM((64, 128), jnp.float32),
        pltpu.VMEM((64, 128), jnp.float32),
    ],
)
def _pack_kernel(tt_hbm, packed_hbm, buf, buft):
    wid = _worker_id()
    iotas = [lax.iota(jnp.int32, 16) + (16 * t) for t in range(4)]

    def transpose_block(n_pairs):
        for p in range(n_pairs):
            for g in range(8):
                col = jnp.full((16,), 2 * p + (g // 4), jnp.int32)
                rows = iotas[g % 4]
                val = plsc.load_gather(buf, [rows, col]) * SCALE
                buft[p, pl.ds(g * 16, 16)] = val

    def blk_body(k, carry):
        c = wid + k * NUM_WORKERS

        @pl.when(c < NBLK)
        def _():
            pltpu.sync_copy(tt_hbm.at[:, pl.ds(c * 128, 128)], buf)
            transpose_block(64)
            pltpu.sync_copy(buft, packed_hbm.at[pl.ds(c * 64, 64)])

        return carry

    lax.fori_loop(0, BLK_PER_W, blk_body, 0)

    @pl.when(wid == NUM_WORKERS - 1)
    def _():
        for d in range(64):
            pltpu.sync_copy(
                tt_hbm.at[d, pl.ds(NBLK * 128, 64)], buf.at[d, pl.ds(0, 64)]
            )
        transpose_block(32)
        pltpu.sync_copy(
            buft.at[pl.ds(0, 32)], packed_hbm.at[pl.ds(NBLK * 64, 32)]
        )


@functools.partial(
    pl.kernel,
    mesh=_MESH,
    out_type=jax.ShapeDtypeStruct((200, 64, 4096), jnp.float32),
    compiler_params=_PARAMS,
    scratch_types=[
        pltpu.VMEM((200, 128), jnp.int32),
        pltpu.VMEM((128,), jnp.int32),
        pltpu.VMEM((128, 128), jnp.float32),
        pltpu.VMEM((64, 128), jnp.float32),
        pltpu.SemaphoreType.DMA,
    ],
)
def _gather_kernel(packed_hbm, xt_hbm, out_hbm, xb, pidx, rows, buft, sem):
    wid = _worker_id()
    bcol = wid * 128
    pltpu.sync_copy(xt_hbm.at[:, pl.ds(bcol, 128)], xb)
    iotas = [lax.iota(jnp.int32, 16) + (16 * t) for t in range(8)]

    def s_body(s, carry):
        halves = []
        for g in range(8):
            sl = pl.ds(g * 16, 16)
            xv = xb[s, sl]
            pidx[sl] = xv >> 1
            halves.append((xv & 1) * 64)
        pltpu.async_copy(packed_hbm.at[pidx], rows, sem).wait()
        for d in range(64):
            for g in range(8):
                val = plsc.load_gather(rows, [iotas[g], halves[g] + d])
                buft[d, pl.ds(g * 16, 16)] = val
        pltpu.sync_copy(buft, out_hbm.at[s, :, pl.ds(bcol, 128)])
        return carry

    lax.fori_loop(0, 200, s_body, 0)


def kernel(x, table):
    packed = _pack_kernel(table.T)
    outp = _gather_kernel(packed, x.T)
    return jnp.transpose(outp, (2, 0, 1))


# speedup vs baseline: 2.1017x; 2.1017x over previous
import functools

import jax
import jax.numpy as jnp
from jax import lax
from jax.experimental import pallas as pl
from jax.experimental.pallas import tpu as pltpu
from jax.experimental.pallas import tpu_sc as plsc

D_MODEL = 64
SCALE = 8.0
NUM_CORES = 2
NUM_SUBCORES = 16
NUM_WORKERS = NUM_CORES * NUM_SUBCORES
VOCAB = 1000000
NPAIR = VOCAB // 2
NBLK = VOCAB // 128
TOTAL_K = 246
SEQ = 200

_MESH = plsc.VectorSubcoreMesh(core_axis_name="c", subcore_axis_name="s")
_PARAMS = pltpu.CompilerParams(
    use_tc_tiling_on_sc=True, needs_layout_passes=False
)


def _worker_id():
    return lax.axis_index("s") * NUM_CORES + lax.axis_index("c")


def _bc16(v):
    return jnp.broadcast_to(v, (16,))


@functools.partial(
    pl.kernel,
    mesh=_MESH,
    out_type=jax.ShapeDtypeStruct((NPAIR, 128), jnp.float32),
    compiler_params=_PARAMS,
    scratch_types=[
        pltpu.VMEM((64, 128), jnp.float32),
        pltpu.VMEM((64, 128), jnp.float32),
        pltpu.VMEM((64, 128), jnp.float32),
        pltpu.VMEM((64, 128), jnp.float32),
        pltpu.SemaphoreType.DMA,
        pltpu.SemaphoreType.DMA,
        pltpu.SemaphoreType.DMA,
        pltpu.SemaphoreType.DMA,
    ],
)
def _pack_kernel(tt_hbm, packed_hbm, buf0, buf1, buft0, buft1,
                 i0, i1, o0, o1):
    wid = _worker_id()
    bufs = (buf0, buf1)
    bufts = (buft0, buft1)
    isem = (i0, i1)
    osem = (o0, o1)
    iota16 = lax.iota(jnp.int32, 16)

    def blk(k):
        return jnp.minimum(wid + k * NUM_WORKERS, NBLK - 1)

    def start_in(k, slot):
        c = blk(k)
        pltpu.async_copy(
            tt_hbm.at[:, pl.ds(c * 128, 128)], bufs[slot], isem[slot]
        )

    def start_out(k, slot):
        c = blk(k)
        pltpu.async_copy(
            bufts[slot], packed_hbm.at[pl.ds(c * 64, 64)], osem[slot]
        )

    def wait_in(slot):
        pltpu.make_async_copy(
            tt_hbm.at[:, pl.ds(0, 128)], bufs[slot], isem[slot]
        ).wait()

    def wait_out(slot):
        pltpu.make_async_copy(
            bufts[slot], packed_hbm.at[pl.ds(0, 64)], osem[slot]
        ).wait()

    def transpose_block(slot, n_pairs):
        buf = bufs[slot]
        buft = bufts[slot]
        for g in range(8):
            rows_g = iota16 + (g % 4) * 16
            h = g // 4
            for p0 in range(0, n_pairs, 8):
                vals = [
                    plsc.load_gather(buf, [rows_g, _bc16(2 * p + h)]) * SCALE
                    for p in range(p0, p0 + 8)
                ]
                for j, p in enumerate(range(p0, p0 + 8)):
                    buft[p, pl.ds(g * 16, 16)] = vals[j]

    def half(k, slot, first):
        wait_in(slot)
        if not first:
            wait_out(slot)
        transpose_block(slot, 64)
        start_out(k, slot)

    start_in(0, 0)
    start_in(1, 1)
    half(0, 0, True)
    start_in(2, 0)
    half(1, 1, True)
    start_in(3, 1)

    def pair(i, carry):
        k = 2 * i
        half(k, 0, False)

        @pl.when(k + 2 < TOTAL_K)
        def _():
            start_in(k + 2, 0)

        half(k + 1, 1, False)

        @pl.when(k + 3 < TOTAL_K)
        def _():
            start_in(k + 3, 1)

        return carry

    lax.fori_loop(1, TOTAL_K // 2, pair, 0)
    wait_out(0)
    wait_out(1)

    @pl.when(wid == NUM_WORKERS - 1)
    def _():
        for d in range(64):
            pltpu.sync_copy(
                tt_hbm.at[d, pl.ds(NBLK * 128, 64)], buf0.at[d, pl.ds(0, 64)]
            )
        transpose_block(0, 32)
        pltpu.sync_copy(
            buft0.at[pl.ds(0, 32)], packed_hbm.at[pl.ds(NBLK * 64, 32)]
        )


@functools.partial(
    pl.kernel,
    mesh=_MESH,
    out_type=jax.ShapeDtypeStruct((SEQ, 64, 4096), jnp.float32),
    compiler_params=_PARAMS,
    scratch_types=[
        pltpu.VMEM((SEQ, 128), jnp.int32),
        pltpu.VMEM((128,), jnp.int32),
        pltpu.VMEM((128,), jnp.int32),
        pltpu.VMEM((128, 128), jnp.float32),
        pltpu.VMEM((128, 128), jnp.float32),
        pltpu.VMEM((64, 128), jnp.float32),
        pltpu.VMEM((64, 128), jnp.float32),
        pltpu.SemaphoreType.DMA,
        pltpu.SemaphoreType.DMA,
        pltpu.SemaphoreType.DMA,
        pltpu.SemaphoreType.DMA,
    ],
)
def _gather_kernel(packed_hbm, xt_hbm, out_hbm, xb, pidx0, pidx1,
                   rows0, rows1, buft0, buft1, g0, g1, o0, o1):
    wid = _worker_id()
    bcol = wid * 128
    pidxs = (pidx0, pidx1)
    rows = (rows0, rows1)
    bufts = (buft0, buft1)
    gsem = (g0, g1)
    osem = (o0, o1)
    iota16 = lax.iota(jnp.int32, 16)
    pltpu.sync_copy(xt_hbm.at[:, pl.ds(bcol, 128)], xb)

    def start_gather(s, slot):
        pidx = pidxs[slot]
        for g in range(8):
            sl = pl.ds(g * 16, 16)
            pidx[sl] = xb[s, sl] >> 1
        pltpu.async_copy(packed_hbm.at[pidx], rows[slot], gsem[slot])

    def wait_gather(slot):
        pltpu.make_async_copy(
            packed_hbm.at[pl.ds(0, 128)], rows[slot], gsem[slot]
        ).wait()

    def start_out(s, slot):
        pltpu.async_copy(
            bufts[slot], out_hbm.at[s, :, pl.ds(bcol, 128)], osem[slot]
        )

    def wait_out(slot):
        pltpu.make_async_copy(
            bufts[slot], out_hbm.at[0, :, pl.ds(bcol, 128)], osem[slot]
        ).wait()

    def transpose(s, slot):
        rv = rows[slot]
        buft = bufts[slot]
        for g in range(8):
            rows_g = iota16 + g * 16
            hv = (xb[s, pl.ds(g * 16, 16)] & 1) * 64
            for d0 in range(0, 64, 8):
                vals = [
                    plsc.load_gather(rv, [rows_g, hv + d])
                    for d in range(d0, d0 + 8)
                ]
                for j, d in enumerate(range(d0, d0 + 8)):
                    buft[d, pl.ds(g * 16, 16)] = vals[j]

    def half(s, slot, first):
        wait_gather(slot)
        if not first:
            wait_out(slot)
        transpose(s, slot)
        start_out(s, slot)

    start_gather(0, 0)
    start_gather(1, 1)
    half(0, 0, True)
    start_gather(2, 0)
    half(1, 1, True)
    start_gather(3, 1)

    def pair(i, carry):
        s = 2 * i
        half(s, 0, False)

        @pl.when(s + 2 < SEQ)
        def _():
            start_gather(s + 2, 0)

        half(s + 1, 1, False)

        @pl.when(s + 3 < SEQ)
        def _():
            start_gather(s + 3, 1)

        return carry

    lax.fori_loop(1, SEQ // 2, pair, 0)
    wait_out(0)
    wait_out(1)


def kernel(x, table):
    packed = _pack_kernel(table.T)
    outp = _gather_kernel(packed, x.T)
    return jnp.transpose(outp, (2, 0, 1))


# speedup vs baseline: 2.1066x; 1.0023x over previous
import functools

import jax
import jax.numpy as jnp
from jax import lax
from jax.experimental import pallas as pl
from jax.experimental.pallas import tpu as pltpu
from jax.experimental.pallas import tpu_sc as plsc

D_MODEL = 64
SCALE = 8.0
NUM_CORES = 2
NUM_SUBCORES = 16
NUM_WORKERS = NUM_CORES * NUM_SUBCORES
VOCAB = 1000000
NPAIR = VOCAB // 2
NBLK = VOCAB // 128
TOTAL_K = 248
SEQ = 200
NBUF = 4

_MESH = plsc.VectorSubcoreMesh(core_axis_name="c", subcore_axis_name="s")
_PARAMS = pltpu.CompilerParams(
    use_tc_tiling_on_sc=True, needs_layout_passes=False
)


def _worker_id():
    return lax.axis_index("s") * NUM_CORES + lax.axis_index("c")


def _bc16(v):
    return jnp.broadcast_to(v, (16,))


@functools.partial(
    pl.kernel,
    mesh=_MESH,
    out_type=jax.ShapeDtypeStruct((NPAIR, 128), jnp.float32),
    compiler_params=_PARAMS,
    scratch_types=(
        [pltpu.VMEM((64, 128), jnp.float32) for _ in range(NBUF)]
        + [pltpu.VMEM((64, 128), jnp.float32) for _ in range(NBUF)]
        + [pltpu.SemaphoreType.DMA for _ in range(2 * NBUF)]
    ),
)
def _pack_kernel(tt_hbm, packed_hbm, *scratch):
    wid = _worker_id()
    bufs = scratch[0:NBUF]
    bufts = scratch[NBUF:2 * NBUF]
    isem = scratch[2 * NBUF:3 * NBUF]
    osem = scratch[3 * NBUF:4 * NBUF]
    iota16 = lax.iota(jnp.int32, 16)

    def blk(k):
        return jnp.minimum(wid + k * NUM_WORKERS, NBLK - 1)

    def start_in(k, slot):
        pltpu.async_copy(
            tt_hbm.at[:, pl.ds(blk(k) * 128, 128)], bufs[slot], isem[slot]
        )

    def start_out(k, slot):
        pltpu.async_copy(
            bufts[slot], packed_hbm.at[pl.ds(blk(k) * 64, 64)], osem[slot]
        )

    def wait_in(slot):
        pltpu.make_async_copy(
            tt_hbm.at[:, pl.ds(0, 128)], bufs[slot], isem[slot]
        ).wait()

    def wait_out(slot):
        pltpu.make_async_copy(
            bufts[slot], packed_hbm.at[pl.ds(0, 64)], osem[slot]
        ).wait()

    def transpose_block(slot, n_pairs):
        buf = bufs[slot]
        buft = bufts[slot]
        for g in range(8):
            rows_g = iota16 + (g % 4) * 16
            h = g // 4

            def pbody(i, carry, rows_g=rows_g, h=h, g=g):
                base = i * 8
                vals = [
                    plsc.load_gather(
                        buf, [rows_g, _bc16(2 * (base + j) + h)]
                    ) * SCALE
                    for j in range(8)
                ]
                for j in range(8):
                    buft[base + j, pl.ds(g * 16, 16)] = vals[j]
                return carry

            lax.fori_loop(0, n_pairs // 8, pbody, 0)

    def half(k, slot, first):
        wait_in(slot)
        if not first:
            wait_out(slot)
        transpose_block(slot, 64)
        start_out(k, slot)

    for k in range(NBUF):
        start_in(k, k)
    for k in range(NBUF):
        half(k, k, True)
        start_in(k + NBUF, k)

    def ring(q, carry):
        for slot in range(NBUF):
            k = NBUF * q + slot
            half(k, slot, False)

            @pl.when(k + NBUF < TOTAL_K)
            def _():
                start_in(k + NBUF, slot)

        return carry

    lax.fori_loop(1, TOTAL_K // NBUF, ring, 0)
    for slot in range(NBUF):
        wait_out(slot)

    @pl.when(wid == NUM_WORKERS - 1)
    def _():
        for d in range(64):
            pltpu.sync_copy(
                tt_hbm.at[d, pl.ds(NBLK * 128, 64)],
                bufs[0].at[d, pl.ds(0, 64)],
            )
        transpose_block(0, 32)
        pltpu.sync_copy(
            bufts[0].at[pl.ds(0, 32)], packed_hbm.at[pl.ds(NBLK * 64, 32)]
        )


@functools.partial(
    pl.kernel,
    mesh=_MESH,
    out_type=jax.ShapeDtypeStruct((SEQ, 64, 4096), jnp.float32),
    compiler_params=_PARAMS,
    scratch_types=(
        [pltpu.VMEM((SEQ, 128), jnp.int32)]
        + [pltpu.VMEM((128,), jnp.int32) for _ in range(NBUF)]
        + [pltpu.VMEM((128, 128), jnp.float32) for _ in range(NBUF)]
        + [pltpu.VMEM((64, 128), jnp.float32) for _ in range(2)]
        + [pltpu.SemaphoreType.DMA for _ in range(NBUF + 2)]
    ),
)
def _gather_kernel(packed_hbm, xt_hbm, out_hbm, xb, *scratch):
    wid = _worker_id()
    bcol = wid * 128
    pidxs = scratch[0:NBUF]
    rows = scratch[NBUF:2 * NBUF]
    bufts = scratch[2 * NBUF:2 * NBUF + 2]
    gsem = scratch[2 * NBUF + 2:3 * NBUF + 2]
    osem = scratch[3 * NBUF + 2:3 * NBUF + 4]
    iota16 = lax.iota(jnp.int32, 16)
    pltpu.sync_copy(xt_hbm.at[:, pl.ds(bcol, 128)], xb)

    def start_gather(s, slot):
        pidx = pidxs[slot]
        for g in range(8):
            sl = pl.ds(g * 16, 16)
            pidx[sl] = xb[s, sl] >> 1
        pltpu.async_copy(packed_hbm.at[pidx], rows[slot], gsem[slot])

    def wait_gather(slot):
        pltpu.make_async_copy(
            packed_hbm.at[pl.ds(0, 128)], rows[slot], gsem[slot]
        ).wait()

    def start_out(s, oslot):
        pltpu.async_copy(
            bufts[oslot], out_hbm.at[s, :, pl.ds(bcol, 128)], osem[oslot]
        )

    def wait_out(oslot):
        pltpu.make_async_copy(
            bufts[oslot], out_hbm.at[0, :, pl.ds(bcol, 128)], osem[oslot]
        ).wait()

    def transpose(s, slot, oslot):
        rv = rows[slot]
        buft = bufts[oslot]
        for g in range(8):
            rows_g = iota16 + g * 16
            hv = (xb[s, pl.ds(g * 16, 16)] & 1) * 64

            def dbody(i, carry, rows_g=rows_g, hv=hv, g=g):
                base = i * 8
                vals = [
                    plsc.load_gather(rv, [rows_g, hv + (base + j)])
                    for j in range(8)
                ]
                for j in range(8):
                    buft[base + j, pl.ds(g * 16, 16)] = vals[j]
                return carry

            lax.fori_loop(0, 8, dbody, 0)

    def half(s, slot, first):
        oslot = slot % 2
        wait_gather(slot)
        if not first:
            wait_out(oslot)
        transpose(s, slot, oslot)
        start_out(s, oslot)

    for s in range(NBUF):
        start_gather(s, s)
    for s in range(2):
        half(s, s, True)
        start_gather(s + NBUF, s)
    for s in range(2, NBUF):
        half(s, s, False)
        start_gather(s + NBUF, s)

    def ring(q, carry):
        for slot in range(NBUF):
            s = NBUF * q + slot
            half(s, slot, False)

            @pl.when(s + NBUF < SEQ)
            def _():
                start_gather(s + NBUF, slot)

        return carry

    lax.fori_loop(1, SEQ // NBUF, ring, 0)
    wait_out(0)
    wait_out(1)


def kernel(x, table):
    packed = _pack_kernel(table.T)
    outp = _gather_kernel(packed, x.T)
    return jnp.transpose(outp, (2, 0, 1))


# speedup vs baseline: 3.0766x; 1.4604x over previous
import functools

import jax
import jax.numpy as jnp
from jax import lax
from jax.experimental import pallas as pl
from jax.experimental.pallas import tpu as pltpu
from jax.experimental.pallas import tpu_sc as plsc

D_MODEL = 64
SCALE = 8.0
NUM_CORES = 2
NUM_SUBCORES = 16
NUM_WORKERS = NUM_CORES * NUM_SUBCORES
VOCAB = 1000000
NPAIR = VOCAB // 2
NBLK = VOCAB // 128
TOTAL_K = 248
SEQ = 200
NBUF = 4

_MESH = plsc.VectorSubcoreMesh(core_axis_name="c", subcore_axis_name="s")
_PARAMS = pltpu.CompilerParams(
    use_tc_tiling_on_sc=True, needs_layout_passes=False
)


def _worker_id():
    return lax.axis_index("s") * NUM_CORES + lax.axis_index("c")


def _bc16(v):
    return jnp.broadcast_to(v, (16,))


@functools.partial(
    pl.kernel,
    mesh=_MESH,
    out_type=jax.ShapeDtypeStruct((NPAIR, 128), jnp.float32),
    compiler_params=_PARAMS,
    scratch_types=(
        [pltpu.VMEM((64, 128), jnp.float32) for _ in range(NBUF)]
        + [pltpu.VMEM((64, 128), jnp.float32) for _ in range(NBUF)]
        + [pltpu.SemaphoreType.DMA for _ in range(2 * NBUF)]
    ),
)
def _pack_kernel(tt_hbm, packed_hbm, *scratch):
    wid = _worker_id()
    bufs = scratch[0:NBUF]
    bufts = scratch[NBUF:2 * NBUF]
    isem = scratch[2 * NBUF:3 * NBUF]
    osem = scratch[3 * NBUF:4 * NBUF]
    iota16 = lax.iota(jnp.int32, 16)

    def blk(k):
        return jnp.minimum(wid + k * NUM_WORKERS, NBLK - 1)

    def start_in(k, slot):
        pltpu.async_copy(
            tt_hbm.at[:, pl.ds(blk(k) * 128, 128)], bufs[slot], isem[slot]
        )

    def start_out(k, slot):
        pltpu.async_copy(
            bufts[slot], packed_hbm.at[pl.ds(blk(k) * 64, 64)], osem[slot]
        )

    def wait_in(slot):
        pltpu.make_async_copy(
            tt_hbm.at[:, pl.ds(0, 128)], bufs[slot], isem[slot]
        ).wait()

    def wait_out(slot):
        pltpu.make_async_copy(
            bufts[slot], packed_hbm.at[pl.ds(0, 64)], osem[slot]
        ).wait()

    halfvec = (iota16 & 1) * 64

    def transpose_block(slot, n_pairs):
        buf = bufs[slot]
        buft = bufts[slot]

        def tbody(t, carry):
            j0 = (t // 4) * 16
            d0v = _bc16((t % 4) * 16)
            colj = iota16 + j0
            pv = colj >> 1
            for k in range(16):
                permk = (iota16 + k) & 15
                rowd = d0v + permk
                c2v = halfvec + permk + d0v
                val = plsc.load_gather(buf, [rowd, colj]) * SCALE
                plsc.store_scatter(buft, [pv, c2v], val)
            return carry

        lax.fori_loop(0, (2 * n_pairs // 16) * 4, tbody, 0)

    def half(k, slot, first):
        wait_in(slot)
        if not first:
            wait_out(slot)
        transpose_block(slot, 64)
        start_out(k, slot)

    for k in range(NBUF):
        start_in(k, k)
    for k in range(NBUF):
        half(k, k, True)
        start_in(k + NBUF, k)

    def ring(q, carry):
        for slot in range(NBUF):
            k = NBUF * q + slot
            half(k, slot, False)

            @pl.when(k + NBUF < TOTAL_K)
            def _():
                start_in(k + NBUF, slot)

        return carry

    lax.fori_loop(1, TOTAL_K // NBUF, ring, 0)
    for slot in range(NBUF):
        wait_out(slot)

    @pl.when(wid == NUM_WORKERS - 1)
    def _():
        for d in range(64):
            pltpu.sync_copy(
                tt_hbm.at[d, pl.ds(NBLK * 128, 64)],
                bufs[0].at[d, pl.ds(0, 64)],
            )
        transpose_block(0, 32)
        pltpu.sync_copy(
            bufts[0].at[pl.ds(0, 32)], packed_hbm.at[pl.ds(NBLK * 64, 32)]
        )


@functools.partial(
    pl.kernel,
    mesh=_MESH,
    out_type=jax.ShapeDtypeStruct((SEQ, 64, 4096), jnp.float32),
    compiler_params=_PARAMS,
    scratch_types=(
        [pltpu.VMEM((SEQ, 128), jnp.int32)]
        + [pltpu.VMEM((128,), jnp.int32) for _ in range(NBUF)]
        + [pltpu.VMEM((128, 128), jnp.float32) for _ in range(NBUF)]
        + [pltpu.VMEM((64, 128), jnp.float32) for _ in range(2)]
        + [pltpu.SemaphoreType.DMA for _ in range(NBUF + 2)]
    ),
)
def _gather_kernel(packed_hbm, xt_hbm, out_hbm, xb, *scratch):
    wid = _worker_id()
    bcol = wid * 128
    pidxs = scratch[0:NBUF]
    rows = scratch[NBUF:2 * NBUF]
    bufts = scratch[2 * NBUF:2 * NBUF + 2]
    gsem = scratch[2 * NBUF + 2:3 * NBUF + 2]
    osem = scratch[3 * NBUF + 2:3 * NBUF + 4]
    iota16 = lax.iota(jnp.int32, 16)
    pltpu.sync_copy(xt_hbm.at[:, pl.ds(bcol, 128)], xb)

    def start_gather(s, slot):
        pidx = pidxs[slot]
        for g in range(8):
            sl = pl.ds(g * 16, 16)
            pidx[sl] = xb[s, sl] >> 1
        pltpu.async_copy(packed_hbm.at[pidx], rows[slot], gsem[slot])

    def wait_gather(slot):
        pltpu.make_async_copy(
            packed_hbm.at[pl.ds(0, 128)], rows[slot], gsem[slot]
        ).wait()

    def start_out(s, oslot):
        pltpu.async_copy(
            bufts[oslot], out_hbm.at[s, :, pl.ds(bcol, 128)], osem[oslot]
        )

    def wait_out(oslot):
        pltpu.make_async_copy(
            bufts[oslot], out_hbm.at[0, :, pl.ds(bcol, 128)], osem[oslot]
        ).wait()

    def transpose(s, slot, oslot):
        rv = rows[slot]
        buft = bufts[oslot]
        def tbody(t, carry):
            j0 = (t // 4) * 16
            d0v = _bc16((t % 4) * 16)
            jv = iota16 + j0
            hv = (xb[s, pl.ds(j0, 16)] & 1) * 64
            for k in range(16):
                permk = (iota16 + k) & 15
                dvec = d0v + permk
                val = plsc.load_gather(rv, [jv, hv + dvec])
                plsc.store_scatter(buft, [dvec, jv], val)
            return carry

        lax.fori_loop(0, 32, tbody, 0)

    def half(s, slot, first):
        oslot = slot % 2
        wait_gather(slot)
        if not first:
            wait_out(oslot)
        transpose(s, slot, oslot)
        start_out(s, oslot)

    for s in range(NBUF):
        start_gather(s, s)
    for s in range(2):
        half(s, s, True)
        start_gather(s + NBUF, s)
    for s in range(2, NBUF):
        half(s, s, False)
        start_gather(s + NBUF, s)

    def ring(q, carry):
        for slot in range(NBUF):
            s = NBUF * q + slot
            half(s, slot, False)

            @pl.when(s + NBUF < SEQ)
            def _():
                start_gather(s + NBUF, slot)

        return carry

    lax.fori_loop(1, SEQ // NBUF, ring, 0)
    wait_out(0)
    wait_out(1)


def kernel(x, table):
    packed = _pack_kernel(table.T)
    outp = _gather_kernel(packed, x.T)
    return jnp.transpose(outp, (2, 0, 1))
